# baseline (speedup 1.0000x reference)
import functools

import jax
import jax.numpy as jnp
from jax import lax
from jax.experimental import pallas as pl
from jax.experimental.pallas import tpu as pltpu
from jax.experimental.pallas import tpu_sc as plsc

BUCKETS = 100000
D = 16
B = 16384
NC = 2
NS = 16
NW = NC * NS
BPW = B // NW
CHUNK = 128
NCHUNK = BPW // CHUNK
ROWS_PER_SUPER = 128 // D
SUPER = BUCKETS // ROWS_PER_SUPER
NBLK = (BUCKETS + 127) // 128
BLK_PER_W = (NBLK + NW - 1) // NW

_SC_PARAMS = pltpu.CompilerParams(
    needs_layout_passes=False, use_tc_tiling_on_sc=True,
    disable_bounds_checks=True)
_MESH = dict(core_axis_name="c", subcore_axis_name="s")


@functools.partial(
    pl.kernel,
    out_type=(jax.ShapeDtypeStruct((SUPER, 128), jnp.float32),
              jax.ShapeDtypeStruct((SUPER, 128), jnp.float32)),
    mesh=plsc.VectorSubcoreMesh(**_MESH),
    compiler_params=_SC_PARAMS,
    scratch_types=[
        pltpu.VMEM((D, 128), jnp.float32),
        pltpu.VMEM((D, 128), jnp.float32),
    ],
)
def _transpose_sc(ut_hbm, it_hbm, su_hbm, st_hbm, vab, vo):
    wid = lax.axis_index("s") * NC + lax.axis_index("c")
    iot = lax.iota(jnp.int32, D)
    nfull = BUCKETS // 128

    def block(i, carry):
        j = jnp.minimum(wid + NW * i, nfull - 1)
        c0 = pl.multiple_of(j * 128, 128)
        rows0 = pl.multiple_of(j * D, 8)
        for src, dst in ((ut_hbm, su_hbm), (it_hbm, st_hbm)):
            pltpu.sync_copy(src.at[pl.ds(0, D), pl.ds(c0, 128)], vab)
            for rr in range(D):
                for r8 in range(ROWS_PER_SUPER):
                    c = rr * ROWS_PER_SUPER + r8
                    y = plsc.load_gather(vab, [iot, jnp.full((D,), c, jnp.int32)])
                    vo[rr, pl.ds(r8 * D, D)] = y
            pltpu.sync_copy(vo, dst.at[pl.ds(rows0, D), :])
        return carry

    lax.fori_loop(0, BLK_PER_W, block, 0)

    @pl.when(wid == 0)
    def _tail():
        tvalid = (BUCKETS - nfull * 128) // ROWS_PER_SUPER
        tc0 = pl.multiple_of(jnp.int32(nfull) * 128, 128)
        trows = pl.multiple_of(jnp.int32(nfull) * D, 8)
        for src, dst in ((ut_hbm, su_hbm), (it_hbm, st_hbm)):
            pltpu.sync_copy(src.at[pl.ds(0, D), pl.ds(tc0, 128)], vab)
            for rr in range(tvalid):
                for r8 in range(ROWS_PER_SUPER):
                    c = rr * ROWS_PER_SUPER + r8
                    y = plsc.load_gather(vab, [iot, jnp.full((D,), c, jnp.int32)])
                    vo[rr, pl.ds(r8 * D, D)] = y
            pltpu.sync_copy(vo.at[pl.ds(0, tvalid), :],
                            dst.at[pl.ds(trows, tvalid), :])


@functools.partial(
    pl.kernel,
    out_type=jax.ShapeDtypeStruct((B,), jnp.float32),
    mesh=plsc.VectorSubcoreMesh(**_MESH),
    compiler_params=_SC_PARAMS,
    scratch_types=[
        pltpu.VMEM((BPW,), jnp.int32),
        pltpu.VMEM((BPW,), jnp.int32),
        pltpu.VMEM((BPW,), jnp.int32),
        pltpu.VMEM((BPW,), jnp.int32),
        pltpu.VMEM((CHUNK, 128), jnp.float32),
        pltpu.VMEM((CHUNK, 128), jnp.float32),
        pltpu.VMEM((CHUNK, 128), jnp.float32),
        pltpu.VMEM((CHUNK, 128), jnp.float32),
        pltpu.VMEM((BPW,), jnp.float32),
        pltpu.VMEM((D,), jnp.float32),
        pltpu.VMEM((D,), jnp.float32),
        pltpu.SemaphoreType.DMA,
        pltpu.SemaphoreType.DMA,
    ],
)
def _fm_sc(uid_hbm, tid_hbm, utab_hbm, itab_hbm, w_hbm, b_hbm, out_hbm,
           idx_u, idx_t, gu, gt, ub0, ub1, tb0, tb1, out_v, w_v, b_v,
           sem0, sem1):
    wid = lax.axis_index("s") * NC + lax.axis_index("c")
    base = wid * BPW

    pltpu.sync_copy(uid_hbm.at[pl.ds(base, BPW)], idx_u)
    pltpu.sync_copy(tid_hbm.at[pl.ds(base, BPW)], idx_t)
    pltpu.sync_copy(w_hbm, w_v)
    pltpu.sync_copy(b_hbm, b_v)

    def prep(k, carry):
        s = pl.ds(k * D, D)
        gu[s] = lax.shift_right_logical(idx_u[s], 3)
        gt[s] = lax.shift_right_logical(idx_t[s], 3)
        return carry

    lax.fori_loop(0, BPW // D, prep, 0)

    ubufs = (ub0, ub1)
    tbufs = (tb0, tb1)
    sems = (sem0, sem1)

    def fire(j):
        s = pl.ds(j * CHUNK, CHUNK)
        hu = pltpu.async_copy(utab_hbm.at[gu.at[s]], ubufs[j % 2], sems[j % 2])
        ht = pltpu.async_copy(itab_hbm.at[gt.at[s]], tbufs[j % 2], sems[j % 2])
        return hu, ht

    w = w_v[...]
    bb = b_v[...]
    iot = lax.iota(jnp.int32, D)

    handles = fire(0)
    for j in range(NCHUNK):
        nxt = fire(j + 1) if j + 1 < NCHUNK else None
        handles[0].wait()
        handles[1].wait()
        ubuf, tbuf = ubufs[j % 2], tbufs[j % 2]

        def group(g, carry):
            rows = g * D + iot
            s = pl.ds(j * CHUNK + g * D, D)
            cu = lax.shift_left(idx_u[s] & 7, 4)
            ct = lax.shift_left(idx_t[s] & 7, 4)
            acc = jnp.zeros((D,), jnp.float32)
            for d in range(D):
                u = plsc.load_gather(ubuf, [rows, cu + d])
                t = plsc.load_gather(tbuf, [rows, ct + d])
                acc = acc + u * t
            z = acc * w + bb
            out_v[s] = 1.0 / (1.0 + jnp.exp(-z))
            return carry

        lax.fori_loop(0, CHUNK // D, group, 0)
        handles = nxt

    pltpu.sync_copy(out_v, out_hbm.at[pl.ds(base, BPW)])


def kernel(f_uid, f_tid, user_table, item_table, W, b):
    uid = f_uid.astype(jnp.int32)
    tid = f_tid.astype(jnp.int32)
    wvec = jnp.broadcast_to(W.astype(jnp.float32).reshape(()), (D,))
    bvec = jnp.broadcast_to(b.astype(jnp.float32).reshape(()), (D,))
    su, st = _transpose_sc(user_table.T, item_table.T)
    y = _fm_sc(uid, tid, su, st, wvec, bvec)
    return y.reshape(B, 1)
